# baseline (speedup 1.0000x reference)
import jax
import jax.numpy as jnp
from jax import lax
from jax.experimental import pallas as pl
from jax.experimental.pallas import tpu as pltpu

_CONF = 0.25
_IOU_T = 0.45
_MAX_DET = 300
_IMG_W = 640.0
_IMG_H = 640.0
_MAX_NMS = 5000
_MAX_WH = 7680.0
_NA = 8400
_NAP = 8448
_NS = 5120
_BLK = 8
_CHUNK = 128
_NCHUNK = _NS // _CHUNK
_NDET = 304
_OMH = 160
_OMW = 160


def _decode_kernel(pred_ref, out_ref):
    cls = pred_ref[4:84, :]
    conf = jnp.max(cls, axis=0, keepdims=True)
    ridx = lax.broadcasted_iota(jnp.int32, (80, _NAP), 0).astype(jnp.float32)
    jf = jnp.min(jnp.where(cls == conf, ridx, 1e9), axis=0, keepdims=True)
    x = pred_ref[0:1, :]
    y = pred_ref[1:2, :]
    w = pred_ref[2:3, :]
    h = pred_ref[3:4, :]
    x1 = x - w * 0.5
    y1 = y - h * 0.5
    x2 = x + w * 0.5
    y2 = y + h * 0.5
    out_ref[0:1, :] = x1
    out_ref[1:2, :] = y1
    out_ref[2:3, :] = x2
    out_ref[3:4, :] = y2
    valid = conf > _CONF
    out_ref[4:5, :] = jnp.where(valid, conf, -jnp.inf)
    out_ref[5:6, :] = jf
    out_ref[6:7, :] = conf
    out_ref[7:8, :] = (x2 - x1) * (y2 - y1)


def _nms_kernel(rows_ref, rows8_ref, sb_ref, kept_ref, al_ref):
    c = pl.program_id(0)

    @pl.when(c == 0)
    def _init():
        al_ref[...] = rows_ref[5:6, :]

    cs = c * _CHUNK
    ch_alive0 = al_ref[:, pl.ds(cs, _CHUNK)]

    @pl.when(jnp.sum(ch_alive0) > 0.0)
    def _process():
        sbc = sb_ref[0]
        cx1 = rows8_ref[0:8, pl.ds(cs, _CHUNK)]
        cy1 = rows8_ref[8:16, pl.ds(cs, _CHUNK)]
        cx2 = rows8_ref[16:24, pl.ds(cs, _CHUNK)]
        cy2 = rows8_ref[24:32, pl.ds(cs, _CHUNK)]
        carea = rows8_ref[32:40, pl.ds(cs, _CHUNK)]
        x1 = rows8_ref[0:8, :]
        y1 = rows8_ref[8:16, :]
        x2 = rows8_ref[16:24, :]
        y2 = rows8_ref[24:32, :]
        area = rows8_ref[32:40, :]
        lane8 = lax.broadcasted_iota(jnp.int32, (1, _BLK), 1)
        lane = lax.broadcasted_iota(jnp.int32, (1, _NS), 1)
        ei = lax.broadcasted_iota(jnp.int32, (_BLK, _BLK), 0)
        ej = lax.broadcasted_iota(jnp.int32, (_BLK, _BLK), 1)
        eye = (ei == ej).astype(jnp.float32)

        for j in range(_CHUNK // _BLK):
            lo = j * _BLK
            arow0 = al_ref[:, pl.ds(cs, _CHUNK)][:, lo:lo + _BLK]

            @pl.when(jnp.sum(arow0) > 0.0)
            def _sub(arow0=arow0, lo=lo, j=j):
                bx1 = sbc[:, 5 * j + 0:5 * j + 1]
                by1 = sbc[:, 5 * j + 1:5 * j + 2]
                bx2 = sbc[:, 5 * j + 2:5 * j + 3]
                by2 = sbc[:, 5 * j + 3:5 * j + 4]
                barea = sbc[:, 5 * j + 4:5 * j + 5]

                rx1 = cx1[:, lo:lo + _BLK]
                ry1 = cy1[:, lo:lo + _BLK]
                rx2 = cx2[:, lo:lo + _BLK]
                ry2 = cy2[:, lo:lo + _BLK]
                rarea = carea[:, lo:lo + _BLK]
                iw8 = jnp.maximum(jnp.minimum(bx2, rx2) - jnp.maximum(bx1, rx1), 0.0)
                ih8 = jnp.maximum(jnp.minimum(by2, ry2) - jnp.maximum(by1, ry1), 0.0)
                inter8 = iw8 * ih8
                iou8 = inter8 / (barea + rarea - inter8 + 1e-7)
                pair8 = (iou8 > _IOU_T).astype(jnp.float32)
                arow = arow0
                for k in range(_BLK):
                    ak = arow[:, k:k + 1]
                    rowk = pair8[k:k + 1, :]
                    supk = rowk * ak * (lane8 > k).astype(jnp.float32)
                    arow = arow * (1.0 - supk)

                acol = jnp.sum(jnp.broadcast_to(arow, (_BLK, _BLK)) * eye,
                               axis=1, keepdims=True)
                dx1 = bx1 + (1.0 - acol) * 1e9

                iw = jnp.maximum(jnp.minimum(bx2, x2) - jnp.maximum(dx1, x1), 0.0)
                ih = jnp.maximum(jnp.minimum(by2, y2) - jnp.maximum(by1, y1), 0.0)
                inter = iw * ih
                iou = inter / (barea + area - inter + 1e-7)
                sup = jnp.max(iou, axis=0, keepdims=True) > _IOU_T
                after = lane >= cs + lo + _BLK
                al_ref[...] = al_ref[...] * (1.0 - (sup & after).astype(jnp.float32))

                cur = al_ref[:, pl.ds(cs, _CHUNK)]
                lane128 = lax.broadcasted_iota(jnp.int32, (1, _CHUNK), 1)
                own = (lane128 >= lo) & (lane128 < lo + _BLK)
                arow_full = jnp.pad(arow, ((0, 0), (lo, _CHUNK - _BLK - lo)))
                al_ref[:, pl.ds(cs, _CHUNK)] = jnp.where(own, arow_full, cur)

    @pl.when(c == _NCHUNK - 1)
    def _fin():
        kept_ref[...] = al_ref[...]


def _mask_kernel(det_ref, coef_ref, proto_ref, mask_ref, box_ref):
    det = det_ref[...]
    x1 = jnp.clip(det[:, 0:1], 0.0, _IMG_W)
    y1 = jnp.clip(det[:, 1:2], 0.0, _IMG_H)
    x2 = jnp.clip(det[:, 2:3], 0.0, _IMG_W)
    y2 = jnp.clip(det[:, 3:4], 0.0, _IMG_H)
    sv = det[:, 6:7] > 0.0
    zero = jnp.zeros((_BLK, 1), jnp.float32)
    box_ref[:, 0:1] = jnp.where(sv, x1, 0.0)
    box_ref[:, 1:2] = jnp.where(sv, y1, 0.0)
    box_ref[:, 2:3] = jnp.where(sv, x2, 0.0)
    box_ref[:, 3:4] = jnp.where(sv, y2, 0.0)
    box_ref[:, 4:5] = jnp.where(sv, det[:, 4:5], 0.0)
    box_ref[:, 5:6] = jnp.where(sv, det[:, 5:6], 0.0)
    box_ref[:, 6:7] = zero
    box_ref[:, 7:8] = zero

    logits = jnp.dot(coef_ref[...], proto_ref[...],
                     preferred_element_type=jnp.float32,
                     precision=lax.Precision.HIGHEST)
    pos = lax.broadcasted_iota(jnp.int32, (_BLK, _OMH * _OMW), 1)
    wc = (pos % _OMW).astype(jnp.float32)
    hr = (pos // _OMW).astype(jnp.float32)
    sx = _OMW / _IMG_W
    sy = _OMH / _IMG_H
    inside = ((wc >= x1 * sx) & (wc < x2 * sx)
              & (hr >= y1 * sy) & (hr < y2 * sy))
    keep = (logits > 0.0) & inside & sv
    mask_ref[...] = keep.astype(jnp.uint8)


def kernel(prediction, proto):
    pred = jnp.asarray(prediction, dtype=jnp.float32)[0]
    prot = jnp.asarray(proto, dtype=jnp.float32)[0]

    pred_pad = jnp.pad(pred, ((0, 0), (0, _NAP - _NA)))
    dec = pl.pallas_call(
        _decode_kernel,
        out_shape=jax.ShapeDtypeStruct((8, _NAP), jnp.float32),
    )(pred_pad)

    vals, order = lax.top_k(dec[4], _MAX_NMS)
    bsel = dec[0:4, :][:, order]
    jf_s = dec[5][order]
    offs = jf_s * _MAX_WH
    ob = bsel + offs[None, :]
    area_s = (ob[2] - ob[0]) * (ob[3] - ob[1])
    valid_s = (vals > _CONF).astype(jnp.float32)
    rows_in = (jnp.zeros((8, _NS), jnp.float32)
               .at[0:4, :_MAX_NMS].set(ob)
               .at[4, :_MAX_NMS].set(area_s)
               .at[5, :_MAX_NMS].set(valid_s))
    sb_in = jnp.pad(
        rows_in[0:5]
        .reshape(5, _NCHUNK, _CHUNK // _BLK, _BLK)
        .transpose(1, 3, 2, 0)
        .reshape(_NCHUNK, _BLK, 80),
        ((0, 0), (0, 0), (0, 48)))

    rows8_in = jnp.repeat(rows_in[0:5], _BLK, axis=0)

    kept = pl.pallas_call(
        _nms_kernel,
        grid=(_NCHUNK,),
        in_specs=[pl.BlockSpec((8, _NS), lambda b: (0, 0)),
                  pl.BlockSpec((5 * _BLK, _NS), lambda b: (0, 0)),
                  pl.BlockSpec((1, _BLK, 128), lambda b: (b, 0, 0))],
        out_specs=pl.BlockSpec((1, _NS), lambda b: (0, 0)),
        out_shape=jax.ShapeDtypeStruct((1, _NS), jnp.float32),
        scratch_shapes=[pltpu.VMEM((1, _NS), jnp.float32)],
    )(rows_in, rows8_in, sb_in)[0, :_MAX_NMS]

    keptb = kept > 0.0
    ki = keptb.astype(jnp.int32)
    rank = jnp.cumsum(ki) - 1
    n = jnp.minimum(jnp.sum(ki), _MAX_DET)
    slot_ok = keptb & (rank < _MAX_DET)
    scatter_idx = jnp.where(slot_ok, rank, _MAX_DET)
    keep_idx = jnp.zeros(_MAX_DET, jnp.int32).at[scatter_idx].set(
        jnp.arange(_MAX_NMS, dtype=jnp.int32), mode="drop")
    slot_valid = (jnp.arange(_MAX_DET) < n).astype(jnp.float32)

    final_idx = order[keep_idx]
    det = (jnp.zeros((_NDET, 8), jnp.float32)
           .at[:_MAX_DET, 0:4].set(dec[0:4, :][:, final_idx].T)
           .at[:_MAX_DET, 4].set(dec[6][final_idx])
           .at[:_MAX_DET, 5].set(dec[5][final_idx])
           .at[:_MAX_DET, 6].set(slot_valid))
    coef = (jnp.zeros((_NDET, 32), jnp.float32)
            .at[:_MAX_DET].set(pred[84:116, :][:, final_idx].T))
    proto_flat = prot.reshape(32, _OMH * _OMW)

    masks, boxout = pl.pallas_call(
        _mask_kernel,
        grid=(_NDET // _BLK,),
        in_specs=[
            pl.BlockSpec((_BLK, 8), lambda i: (i, 0)),
            pl.BlockSpec((_BLK, 32), lambda i: (i, 0)),
            pl.BlockSpec((32, _OMH * _OMW), lambda i: (0, 0)),
        ],
        out_specs=[
            pl.BlockSpec((_BLK, _OMH * _OMW), lambda i: (i, 0)),
            pl.BlockSpec((_BLK, 8), lambda i: (i, 0)),
        ],
        out_shape=[
            jax.ShapeDtypeStruct((_NDET, _OMH * _OMW), jnp.uint8),
            jax.ShapeDtypeStruct((_NDET, 8), jnp.float32),
        ],
    )(det, coef, proto_flat)

    boxes_out = boxout[:_MAX_DET, 0:4][None]
    scores_out = boxout[:_MAX_DET, 4:5][None]
    label_out = boxout[:_MAX_DET, 5:6][None]
    masks_out = masks[:_MAX_DET].reshape(1, _MAX_DET, _OMH, _OMW)
    return (boxes_out, scores_out, label_out, masks_out)


# speedup vs baseline: 1.1400x; 1.1400x over previous
import jax
import jax.numpy as jnp
from jax import lax
from jax.experimental import pallas as pl
from jax.experimental.pallas import tpu as pltpu

_CONF = 0.25
_IOU_T = 0.45
_MAX_DET = 300
_IMG_W = 640.0
_IMG_H = 640.0
_MAX_NMS = 5000
_MAX_WH = 7680.0
_NA = 8400
_NAP = 8448
_NS = 5120
_BLK = 8
_CHUNK = 128
_NCHUNK = _NS // _CHUNK
_NDET = 304
_OMH = 160
_OMW = 160


def _decode_kernel(pred_ref, out_ref):
    cls = pred_ref[4:84, :]
    conf = jnp.max(cls, axis=0, keepdims=True)
    ridx = lax.broadcasted_iota(jnp.int32, (80, _NAP), 0).astype(jnp.float32)
    jf = jnp.min(jnp.where(cls == conf, ridx, 1e9), axis=0, keepdims=True)
    x = pred_ref[0:1, :]
    y = pred_ref[1:2, :]
    w = pred_ref[2:3, :]
    h = pred_ref[3:4, :]
    x1 = x - w * 0.5
    y1 = y - h * 0.5
    x2 = x + w * 0.5
    y2 = y + h * 0.5
    out_ref[0:1, :] = x1
    out_ref[1:2, :] = y1
    out_ref[2:3, :] = x2
    out_ref[3:4, :] = y2
    valid = conf > _CONF
    out_ref[4:5, :] = jnp.where(valid, conf, -jnp.inf)
    out_ref[5:6, :] = jf
    out_ref[6:7, :] = conf
    out_ref[7:8, :] = (x2 - x1) * (y2 - y1)


def _nms_kernel(rows_ref, sb_ref, kept_ref, al_ref):
    c = pl.program_id(0)

    @pl.when(c == 0)
    def _init():
        al_ref[...] = rows_ref[5:6, :]

    cs = c * _CHUNK
    ch_alive0 = al_ref[:, pl.ds(cs, _CHUNK)]

    @pl.when(jnp.sum(ch_alive0) > 0.0)
    def _process():
        sbc = sb_ref[0]
        cx1 = rows_ref[0:1, pl.ds(cs, _CHUNK)]
        cy1 = rows_ref[1:2, pl.ds(cs, _CHUNK)]
        cx2 = rows_ref[2:3, pl.ds(cs, _CHUNK)]
        cy2 = rows_ref[3:4, pl.ds(cs, _CHUNK)]
        carea = rows_ref[4:5, pl.ds(cs, _CHUNK)]
        x1 = rows_ref[0:1, :]
        y1 = rows_ref[1:2, :]
        x2 = rows_ref[2:3, :]
        y2 = rows_ref[3:4, :]
        area = rows_ref[4:5, :]
        lane8 = lax.broadcasted_iota(jnp.int32, (1, _BLK), 1)
        lane = lax.broadcasted_iota(jnp.int32, (1, _NS), 1)
        ei = lax.broadcasted_iota(jnp.int32, (_BLK, _BLK), 0)
        ej = lax.broadcasted_iota(jnp.int32, (_BLK, _BLK), 1)
        eye = (ei == ej).astype(jnp.float32)

        for j in range(_CHUNK // _BLK):
            lo = j * _BLK
            arow0 = al_ref[:, pl.ds(cs, _CHUNK)][:, lo:lo + _BLK]

            def _sub(arow0=arow0, lo=lo, j=j):
                bx1 = sbc[:, 5 * j + 0:5 * j + 1]
                by1 = sbc[:, 5 * j + 1:5 * j + 2]
                bx2 = sbc[:, 5 * j + 2:5 * j + 3]
                by2 = sbc[:, 5 * j + 3:5 * j + 4]
                barea = sbc[:, 5 * j + 4:5 * j + 5]

                rx1 = cx1[:, lo:lo + _BLK]
                ry1 = cy1[:, lo:lo + _BLK]
                rx2 = cx2[:, lo:lo + _BLK]
                ry2 = cy2[:, lo:lo + _BLK]
                rarea = carea[:, lo:lo + _BLK]
                iw8 = jnp.maximum(jnp.minimum(bx2, rx2) - jnp.maximum(bx1, rx1), 0.0)
                ih8 = jnp.maximum(jnp.minimum(by2, ry2) - jnp.maximum(by1, ry1), 0.0)
                inter8 = iw8 * ih8
                iou8 = inter8 / (barea + rarea - inter8 + 1e-7)
                pair8 = (iou8 > _IOU_T).astype(jnp.float32)
                arow = arow0
                for k in range(_BLK):
                    ak = arow[:, k:k + 1]
                    rowk = pair8[k:k + 1, :]
                    supk = rowk * ak * (lane8 > k).astype(jnp.float32)
                    arow = arow * (1.0 - supk)

                acol = jnp.sum(jnp.broadcast_to(arow, (_BLK, _BLK)) * eye,
                               axis=1, keepdims=True)
                dx1 = bx1 + (1.0 - acol) * 1e9

                iw = jnp.maximum(jnp.minimum(bx2, x2) - jnp.maximum(dx1, x1), 0.0)
                ih = jnp.maximum(jnp.minimum(by2, y2) - jnp.maximum(by1, y1), 0.0)
                inter = iw * ih
                iou = inter / (barea + area - inter + 1e-7)
                sup = jnp.max(iou, axis=0, keepdims=True) > _IOU_T
                after = lane >= cs + lo + _BLK
                al_ref[...] = al_ref[...] * (1.0 - (sup & after).astype(jnp.float32))

                cur = al_ref[:, pl.ds(cs, _CHUNK)]
                lane128 = lax.broadcasted_iota(jnp.int32, (1, _CHUNK), 1)
                own = (lane128 >= lo) & (lane128 < lo + _BLK)
                arow_full = jnp.pad(arow, ((0, 0), (lo, _CHUNK - _BLK - lo)))
                al_ref[:, pl.ds(cs, _CHUNK)] = jnp.where(own, arow_full, cur)

            _sub()

    @pl.when(c == _NCHUNK - 1)
    def _fin():
        kept_ref[...] = al_ref[...]


def _mask_kernel(det_ref, coef_ref, proto_ref, mask_ref, box_ref):
    det = det_ref[...]
    x1 = jnp.clip(det[:, 0:1], 0.0, _IMG_W)
    y1 = jnp.clip(det[:, 1:2], 0.0, _IMG_H)
    x2 = jnp.clip(det[:, 2:3], 0.0, _IMG_W)
    y2 = jnp.clip(det[:, 3:4], 0.0, _IMG_H)
    sv = det[:, 6:7] > 0.0
    zero = jnp.zeros((_BLK, 1), jnp.float32)
    box_ref[:, 0:1] = jnp.where(sv, x1, 0.0)
    box_ref[:, 1:2] = jnp.where(sv, y1, 0.0)
    box_ref[:, 2:3] = jnp.where(sv, x2, 0.0)
    box_ref[:, 3:4] = jnp.where(sv, y2, 0.0)
    box_ref[:, 4:5] = jnp.where(sv, det[:, 4:5], 0.0)
    box_ref[:, 5:6] = jnp.where(sv, det[:, 5:6], 0.0)
    box_ref[:, 6:7] = zero
    box_ref[:, 7:8] = zero

    logits = jnp.dot(coef_ref[...], proto_ref[...],
                     preferred_element_type=jnp.float32,
                     precision=lax.Precision.HIGHEST)
    pos = lax.broadcasted_iota(jnp.int32, (_BLK, _OMH * _OMW), 1)
    wc = (pos % _OMW).astype(jnp.float32)
    hr = (pos // _OMW).astype(jnp.float32)
    sx = _OMW / _IMG_W
    sy = _OMH / _IMG_H
    inside = ((wc >= x1 * sx) & (wc < x2 * sx)
              & (hr >= y1 * sy) & (hr < y2 * sy))
    keep = (logits > 0.0) & inside & sv
    mask_ref[...] = keep.astype(jnp.uint8)


def kernel(prediction, proto):
    pred = jnp.asarray(prediction, dtype=jnp.float32)[0]
    prot = jnp.asarray(proto, dtype=jnp.float32)[0]

    pred_pad = jnp.pad(pred, ((0, 0), (0, _NAP - _NA)))
    dec = pl.pallas_call(
        _decode_kernel,
        out_shape=jax.ShapeDtypeStruct((8, _NAP), jnp.float32),
    )(pred_pad)

    vals, order = lax.top_k(dec[4], _MAX_NMS)
    bsel = dec[0:4, :][:, order]
    jf_s = dec[5][order]
    offs = jf_s * _MAX_WH
    ob = bsel + offs[None, :]
    area_s = (ob[2] - ob[0]) * (ob[3] - ob[1])
    valid_s = (vals > _CONF).astype(jnp.float32)
    rows_in = (jnp.zeros((8, _NS), jnp.float32)
               .at[0:4, :_MAX_NMS].set(ob)
               .at[4, :_MAX_NMS].set(area_s)
               .at[5, :_MAX_NMS].set(valid_s))
    sb_in = jnp.pad(
        rows_in[0:5]
        .reshape(5, _NCHUNK, _CHUNK // _BLK, _BLK)
        .transpose(1, 3, 2, 0)
        .reshape(_NCHUNK, _BLK, 80),
        ((0, 0), (0, 0), (0, 48)))

    kept = pl.pallas_call(
        _nms_kernel,
        grid=(_NCHUNK,),
        in_specs=[pl.BlockSpec((8, _NS), lambda b: (0, 0)),
                  pl.BlockSpec((1, _BLK, 128), lambda b: (b, 0, 0))],
        out_specs=pl.BlockSpec((1, _NS), lambda b: (0, 0)),
        out_shape=jax.ShapeDtypeStruct((1, _NS), jnp.float32),
        scratch_shapes=[pltpu.VMEM((1, _NS), jnp.float32)],
    )(rows_in, sb_in)[0, :_MAX_NMS]

    keptb = kept > 0.0
    ki = keptb.astype(jnp.int32)
    rank = jnp.cumsum(ki) - 1
    n = jnp.minimum(jnp.sum(ki), _MAX_DET)
    slot_ok = keptb & (rank < _MAX_DET)
    scatter_idx = jnp.where(slot_ok, rank, _MAX_DET)
    keep_idx = jnp.zeros(_MAX_DET, jnp.int32).at[scatter_idx].set(
        jnp.arange(_MAX_NMS, dtype=jnp.int32), mode="drop")
    slot_valid = (jnp.arange(_MAX_DET) < n).astype(jnp.float32)

    final_idx = order[keep_idx]
    det = (jnp.zeros((_NDET, 8), jnp.float32)
           .at[:_MAX_DET, 0:4].set(dec[0:4, :][:, final_idx].T)
           .at[:_MAX_DET, 4].set(dec[6][final_idx])
           .at[:_MAX_DET, 5].set(dec[5][final_idx])
           .at[:_MAX_DET, 6].set(slot_valid))
    coef = (jnp.zeros((_NDET, 32), jnp.float32)
            .at[:_MAX_DET].set(pred[84:116, :][:, final_idx].T))
    proto_flat = prot.reshape(32, _OMH * _OMW)

    masks, boxout = pl.pallas_call(
        _mask_kernel,
        grid=(_NDET // _BLK,),
        in_specs=[
            pl.BlockSpec((_BLK, 8), lambda i: (i, 0)),
            pl.BlockSpec((_BLK, 32), lambda i: (i, 0)),
            pl.BlockSpec((32, _OMH * _OMW), lambda i: (0, 0)),
        ],
        out_specs=[
            pl.BlockSpec((_BLK, _OMH * _OMW), lambda i: (i, 0)),
            pl.BlockSpec((_BLK, 8), lambda i: (i, 0)),
        ],
        out_shape=[
            jax.ShapeDtypeStruct((_NDET, _OMH * _OMW), jnp.uint8),
            jax.ShapeDtypeStruct((_NDET, 8), jnp.float32),
        ],
    )(det, coef, proto_flat)

    boxes_out = boxout[:_MAX_DET, 0:4][None]
    scores_out = boxout[:_MAX_DET, 4:5][None]
    label_out = boxout[:_MAX_DET, 5:6][None]
    masks_out = masks[:_MAX_DET].reshape(1, _MAX_DET, _OMH, _OMW)
    return (boxes_out, scores_out, label_out, masks_out)
